# baseline (speedup 1.0000x reference)
import jax
import jax.numpy as jnp
from jax import lax
from jax.experimental import pallas as pl
from jax.experimental.pallas import tpu as pltpu
from jax.experimental.pallas import tpu_sc as plsc

VOCAB = 1000000
DIM = 64
BATCH = 4096
SEQ = 200
LANES = 16
VPR = DIM // LANES

N = BATCH * SEQ
NUM_WORKERS = 32
PER_W = N // NUM_WORKERS

CHUNK = SEQ
TPC = CHUNK // 8
NCHUNKS = PER_W // CHUNK


def _gather_body(idx_hbm, scratch_hbm, out_hbm, idx_v, rows0, rows1,
                 nar0, nar1, gsem0, gsem1, wsem0, wsem1):
    wid = lax.axis_index("s") * 2 + lax.axis_index("c")
    base = wid * PER_W
    batch0 = wid * (BATCH // NUM_WORKERS)
    pltpu.sync_copy(idx_hbm.at[pl.ds(base, PER_W)], idx_v)

    rows = (rows0, rows1)
    nars = (nar0, nar1)
    gsems = (gsem0, gsem1)
    wsems = (wsem0, wsem1)

    def start_gather(g, b):
        idx_slice = idx_v.at[pl.ds(g * CHUNK, CHUNK)]
        pltpu.async_copy(scratch_hbm.at[idx_slice], rows[b], gsems[b])

    def wait_gather(b):
        pltpu.make_async_copy(
            scratch_hbm.at[idx_v.at[pl.ds(0, CHUNK)]], rows[b], gsems[b]
        ).wait()

    def start_write(g, b):
        pltpu.async_copy(nars[b], out_hbm.at[batch0 + g], wsems[b])

    def wait_write(b):
        pltpu.make_async_copy(nars[b], out_hbm.at[batch0], wsems[b]).wait()

    def narrow(b):
        @pl.loop(0, TPC)
        def _tile(t):
            for s in range(8):
                for j in range(VPR):
                    nars[b][t, s, pl.ds(j * LANES, LANES)] = (
                        rows[b][t * 8 + s, pl.ds(j * LANES, LANES)])

    def half(g, a, bb):
        @pl.when(g + 1 < NCHUNKS)
        def _():
            start_gather(g + 1, bb)
        wait_gather(a)

        @pl.when(g >= 2)
        def _():
            wait_write(a)
        narrow(a)
        start_write(g, a)

    start_gather(0, 0)

    @pl.loop(0, NCHUNKS, step=2)
    def _chunk(g):
        half(g, 0, 1)
        half(g + 1, 1, 0)

    wait_write(0)
    wait_write(1)


_SC_MESH = plsc.VectorSubcoreMesh(core_axis_name="c", subcore_axis_name="s")


@jax.jit
def _embed(x_flat, table):
    widen = jnp.eye(DIM, 2 * DIM, dtype=jnp.float32)
    scratch = jnp.dot(table, widen, precision=jax.lax.Precision.HIGHEST)
    gather = pl.kernel(
        _gather_body,
        out_type=jax.ShapeDtypeStruct((BATCH, SEQ // 8, 8, DIM), jnp.float32),
        mesh=_SC_MESH,
        scratch_types=[
            pltpu.VMEM((PER_W,), jnp.int32),
            pltpu.VMEM((CHUNK, 2 * DIM), jnp.float32),
            pltpu.VMEM((CHUNK, 2 * DIM), jnp.float32),
            pltpu.VMEM((TPC, 8, DIM), jnp.float32),
            pltpu.VMEM((TPC, 8, DIM), jnp.float32),
            pltpu.SemaphoreType.DMA,
            pltpu.SemaphoreType.DMA,
            pltpu.SemaphoreType.DMA,
            pltpu.SemaphoreType.DMA,
        ],
    )
    return gather(x_flat, scratch)


def kernel(X, table):
    out = _embed(X.reshape(-1), table)
    return out.reshape(BATCH, SEQ, DIM)
